# baseline (speedup 1.0000x reference)
import functools

import jax
import jax.numpy as jnp
from jax.experimental import pallas as pl


def _rvq_voco_body(x_ref, cb_ref, cbt_ref, w1_ref, b1_ref, w2_ref, b2_ref,
                   out_ref, *, q_stages, k_size):
    x = x_ref[...]
    tm, d = x.shape
    res = x
    qout = jnp.zeros_like(x)
    ones_row = jnp.ones((1, d), dtype=jnp.float32)
    for q in range(q_stages):
        cb = cb_ref[q]
        cbt = cbt_ref[q]
        c_sq = jax.lax.dot_general(
            ones_row, cbt * cbt, (((1,), (0,)), ((), ())),
            preferred_element_type=jnp.float32,
            precision=jax.lax.Precision.HIGHEST)
        r_sq = jnp.sum(res * res, axis=1, keepdims=True)
        cross = jax.lax.dot_general(
            res, cbt, (((1,), (0,)), ((), ())),
            preferred_element_type=jnp.float32)
        dist = r_sq - 2.0 * cross + c_sq
        m = jnp.min(dist, axis=1, keepdims=True)
        lane = jax.lax.broadcasted_iota(jnp.int32, dist.shape, 1)
        idx = jnp.min(jnp.where(dist == m, lane, k_size), axis=1, keepdims=True)
        onehot = (lane == idx).astype(jnp.float32)
        quant = jax.lax.dot_general(
            onehot, cb, (((1,), (0,)), ((), ())),
            preferred_element_type=jnp.float32,
            precision=jax.lax.Precision.HIGHEST)
        qout = qout + quant
        res = res - quant
    qout = x + (qout - x)
    feat = jnp.dot(qout, w1_ref[...], preferred_element_type=jnp.float32)
    feat = jax.nn.gelu(feat + b1_ref[...])
    frames = jnp.dot(feat, w2_ref[...], preferred_element_type=jnp.float32)
    out_ref[...] = frames + b2_ref[...]


def kernel(latents, codebook, W1, b1, W2, b2):
    B, N, D = latents.shape
    Q, K, _ = codebook.shape
    DF = W1.shape[1]
    HOP = W2.shape[1]
    BN = B * N
    TM = 512
    x = latents.reshape(BN, D)
    cbt = jnp.swapaxes(codebook, 1, 2)
    out = pl.pallas_call(
        functools.partial(_rvq_voco_body, q_stages=Q, k_size=K),
        grid=(BN // TM,),
        in_specs=[
            pl.BlockSpec((TM, D), lambda i: (i, 0)),
            pl.BlockSpec((Q, K, D), lambda i: (0, 0, 0)),
            pl.BlockSpec((Q, D, K), lambda i: (0, 0, 0)),
            pl.BlockSpec((D, DF), lambda i: (0, 0)),
            pl.BlockSpec((1, DF), lambda i: (0, 0)),
            pl.BlockSpec((DF, HOP), lambda i: (0, 0)),
            pl.BlockSpec((1, HOP), lambda i: (0, 0)),
        ],
        out_specs=pl.BlockSpec((TM, HOP), lambda i: (i, 0)),
        out_shape=jax.ShapeDtypeStruct((BN, HOP), jnp.float32),
    )(x, codebook, cbt, W1, b1.reshape(1, DF), W2, b2.reshape(1, HOP))
    return out.reshape(B, N * HOP)


# speedup vs baseline: 1.5928x; 1.5928x over previous
import functools

import jax
import jax.numpy as jnp
from jax.experimental import pallas as pl
from jax.experimental.pallas import tpu as pltpu


def _rvq_voco_body(x_ref, cb_ref, cbt_ref, w1_ref, b1_ref, w2_ref, b2_ref,
                   out_ref, csq_ref, hi_ref, mid_ref, lo_ref,
                   *, q_stages, k_size):
    @pl.when(pl.program_id(0) == 0)
    def _precompute():
        ones_row = jnp.ones((1, x_ref.shape[1]), dtype=jnp.float32)
        for q in range(q_stages):
            cb = cb_ref[q]
            cbt = cbt_ref[q]
            csq_ref[q:q + 1, :] = jax.lax.dot_general(
                ones_row, cbt * cbt, (((1,), (0,)), ((), ())),
                preferred_element_type=jnp.float32,
                precision=jax.lax.Precision.HIGHEST)
            hi = cb.astype(jnp.bfloat16)
            r1 = cb - hi.astype(jnp.float32)
            mid = r1.astype(jnp.bfloat16)
            r2 = r1 - mid.astype(jnp.float32)
            hi_ref[q] = hi
            mid_ref[q] = mid
            lo_ref[q] = r2.astype(jnp.bfloat16)

    x = x_ref[...]
    res = x
    qout = jnp.zeros_like(x)
    lane = jax.lax.broadcasted_iota(jnp.int32, (x.shape[0], k_size), 1)
    for q in range(q_stages):
        r_sq = jnp.sum(res * res, axis=1, keepdims=True)
        cross = jax.lax.dot_general(
            res, cbt_ref[q], (((1,), (0,)), ((), ())),
            preferred_element_type=jnp.float32)
        dist = r_sq - 2.0 * cross + csq_ref[q:q + 1, :]
        m = jnp.min(dist, axis=1, keepdims=True)
        idx = jnp.min(jnp.where(dist == m, lane, k_size), axis=1, keepdims=True)
        onehot = (lane == idx).astype(jnp.bfloat16)
        dn = (((1,), (0,)), ((), ()))
        quant = (jax.lax.dot_general(onehot, hi_ref[q], dn,
                                     preferred_element_type=jnp.float32)
                 + jax.lax.dot_general(onehot, mid_ref[q], dn,
                                       preferred_element_type=jnp.float32)
                 + jax.lax.dot_general(onehot, lo_ref[q], dn,
                                       preferred_element_type=jnp.float32))
        qout = qout + quant
        res = res - quant
    qout = x + (qout - x)
    feat = jnp.dot(qout, w1_ref[...], preferred_element_type=jnp.float32)
    feat = jax.nn.gelu(feat + b1_ref[...])
    frames = jnp.dot(feat, w2_ref[...], preferred_element_type=jnp.float32)
    out_ref[...] = frames + b2_ref[...]


def kernel(latents, codebook, W1, b1, W2, b2):
    B, N, D = latents.shape
    Q, K, _ = codebook.shape
    DF = W1.shape[1]
    HOP = W2.shape[1]
    BN = B * N
    TM = 512
    x = latents.reshape(BN, D)
    cbt = jnp.swapaxes(codebook, 1, 2)
    out = pl.pallas_call(
        functools.partial(_rvq_voco_body, q_stages=Q, k_size=K),
        grid=(BN // TM,),
        in_specs=[
            pl.BlockSpec((TM, D), lambda i: (i, 0)),
            pl.BlockSpec((Q, K, D), lambda i: (0, 0, 0)),
            pl.BlockSpec((Q, D, K), lambda i: (0, 0, 0)),
            pl.BlockSpec((D, DF), lambda i: (0, 0)),
            pl.BlockSpec((1, DF), lambda i: (0, 0)),
            pl.BlockSpec((DF, HOP), lambda i: (0, 0)),
            pl.BlockSpec((1, HOP), lambda i: (0, 0)),
        ],
        out_specs=pl.BlockSpec((TM, HOP), lambda i: (i, 0)),
        out_shape=jax.ShapeDtypeStruct((BN, HOP), jnp.float32),
        scratch_shapes=[
            pltpu.VMEM((Q, K), jnp.float32),
            pltpu.VMEM((Q, K, D), jnp.bfloat16),
            pltpu.VMEM((Q, K, D), jnp.bfloat16),
            pltpu.VMEM((Q, K, D), jnp.bfloat16),
        ],
    )(x, codebook, cbt, W1, b1.reshape(1, DF), W2, b2.reshape(1, HOP))
    return out.reshape(B, N * HOP)


# speedup vs baseline: 2.7818x; 1.7465x over previous
import functools

import jax
import jax.numpy as jnp
from jax.experimental import pallas as pl
from jax.experimental.pallas import tpu as pltpu


def _rvq_voco_body(x_ref, cb_ref, cbt_ref, w1_ref, b1_ref, w2_ref, b2_ref,
                   out_ref, csq_ref, hml_ref, *, q_stages, k_size):
    @pl.when(pl.program_id(0) == 0)
    def _precompute():
        ones_row = jnp.ones((1, x_ref.shape[1]), dtype=jnp.float32)
        for q in range(q_stages):
            cb = cb_ref[q]
            cbt = cbt_ref[q]
            csq_ref[q:q + 1, :] = jax.lax.dot_general(
                ones_row, cbt * cbt, (((1,), (0,)), ((), ())),
                preferred_element_type=jnp.float32,
                precision=jax.lax.Precision.HIGHEST)
            hi = cb.astype(jnp.bfloat16)
            r1 = cb - hi.astype(jnp.float32)
            mid = r1.astype(jnp.bfloat16)
            r2 = r1 - mid.astype(jnp.float32)
            hml_ref[q] = jnp.concatenate(
                [hi, mid, r2.astype(jnp.bfloat16)], axis=1)

    x = x_ref[...]
    tm, d = x.shape
    res = x
    qout = jnp.zeros_like(x)
    lane = jax.lax.broadcasted_iota(jnp.int32, (tm, k_size), 1)
    for q in range(q_stages):
        r_sq = jnp.sum(res * res, axis=1, keepdims=True)
        cross = jax.lax.dot_general(
            res, cbt_ref[q], (((1,), (0,)), ((), ())),
            preferred_element_type=jnp.float32)
        dist = r_sq - 2.0 * cross + csq_ref[q:q + 1, :]
        code = jnp.argmin(dist, axis=1)
        onehot = (lane == code[:, None]).astype(jnp.bfloat16)
        q3 = jax.lax.dot_general(
            onehot, hml_ref[q], (((1,), (0,)), ((), ())),
            preferred_element_type=jnp.float32)
        quant = (q3[:, :d] + q3[:, d:2 * d]) + q3[:, 2 * d:]
        qout = qout + quant
        res = res - quant
    qout = x + (qout - x)
    feat = jnp.dot(qout, w1_ref[...], preferred_element_type=jnp.float32)
    feat = jax.nn.gelu(feat + b1_ref[...])
    frames = jnp.dot(feat, w2_ref[...], preferred_element_type=jnp.float32)
    out_ref[...] = frames + b2_ref[...]


def kernel(latents, codebook, W1, b1, W2, b2):
    B, N, D = latents.shape
    Q, K, _ = codebook.shape
    DF = W1.shape[1]
    HOP = W2.shape[1]
    BN = B * N
    TM = 512
    x = latents.reshape(BN, D)
    cbt = jnp.swapaxes(codebook, 1, 2)
    out = pl.pallas_call(
        functools.partial(_rvq_voco_body, q_stages=Q, k_size=K),
        grid=(BN // TM,),
        in_specs=[
            pl.BlockSpec((TM, D), lambda i: (i, 0)),
            pl.BlockSpec((Q, K, D), lambda i: (0, 0, 0)),
            pl.BlockSpec((Q, D, K), lambda i: (0, 0, 0)),
            pl.BlockSpec((D, DF), lambda i: (0, 0)),
            pl.BlockSpec((1, DF), lambda i: (0, 0)),
            pl.BlockSpec((DF, HOP), lambda i: (0, 0)),
            pl.BlockSpec((1, HOP), lambda i: (0, 0)),
        ],
        out_specs=pl.BlockSpec((TM, HOP), lambda i: (i, 0)),
        out_shape=jax.ShapeDtypeStruct((BN, HOP), jnp.float32),
        scratch_shapes=[
            pltpu.VMEM((Q, K), jnp.float32),
            pltpu.VMEM((Q, K, 3 * D), jnp.bfloat16),
        ],
    )(x, codebook, cbt, W1, b1.reshape(1, DF), W2, b2.reshape(1, HOP))
    return out.reshape(B, N * HOP)


# speedup vs baseline: 3.2719x; 1.1762x over previous
import functools

import jax
import jax.numpy as jnp
from jax.experimental import pallas as pl
from jax.experimental.pallas import tpu as pltpu


def _rvq_stages(x, cbt_ref, csq_ref, hml_ref, q_stages, lane):
    d = x.shape[1]
    res = x
    qout = jnp.zeros_like(x)
    for q in range(q_stages):
        r_sq = jnp.sum(res * res, axis=1, keepdims=True)
        cross2 = jax.lax.dot_general(
            res, cbt_ref[q], (((1,), (0,)), ((), ())),
            preferred_element_type=jnp.float32)
        dist = (r_sq - cross2) + csq_ref[q:q + 1, :]
        code = jnp.argmin(dist, axis=1)
        sel = (lane == code[:, None]).astype(jnp.bfloat16)
        q3 = jax.lax.dot_general(
            sel, hml_ref[q], (((1,), (0,)), ((), ())),
            preferred_element_type=jnp.float32)
        quant = (q3[:, :d] + q3[:, d:2 * d]) + q3[:, 2 * d:]
        qout = qout + quant
        res = res - quant
    return x + (qout - x)


def _rvq_voco_body(x_ref, cb_ref, cbt_ref, w1_ref, b1_ref, w2_ref, b2_ref,
                   out_ref, csq_ref, hml_ref, *, q_stages):
    @pl.when(pl.program_id(0) == 0)
    def _precompute():
        ones_row = jnp.ones((1, x_ref.shape[1]), dtype=jnp.float32)
        for q in range(q_stages):
            cb = cb_ref[q]
            cbt = cbt_ref[q]
            csq_ref[q:q + 1, :] = 0.25 * jax.lax.dot_general(
                ones_row, cbt * cbt, (((1,), (0,)), ((), ())),
                preferred_element_type=jnp.float32,
                precision=jax.lax.Precision.HIGHEST)
            hi = cb.astype(jnp.bfloat16)
            r1 = cb - hi.astype(jnp.float32)
            mid = r1.astype(jnp.bfloat16)
            r2 = r1 - mid.astype(jnp.float32)
            hml_ref[q] = jnp.concatenate(
                [hi, mid, r2.astype(jnp.bfloat16)], axis=1)

    x = x_ref[...]
    tm = x.shape[0]
    lane = jax.lax.broadcasted_iota(jnp.int32, (tm, csq_ref.shape[1]), 1)
    qout = _rvq_stages(x, cbt_ref, csq_ref, hml_ref, q_stages, lane)
    feat = jnp.dot(qout, w1_ref[...], preferred_element_type=jnp.float32)
    feat = jax.nn.gelu(feat + b1_ref[...])
    frames = jnp.dot(feat, w2_ref[...], preferred_element_type=jnp.float32)
    out_ref[...] = frames + b2_ref[...]


def kernel(latents, codebook, W1, b1, W2, b2):
    B, N, D = latents.shape
    Q, K, _ = codebook.shape
    DF = W1.shape[1]
    HOP = W2.shape[1]
    BN = B * N
    TM = 1024
    x = latents.reshape(BN, D)
    cbt = 2.0 * jnp.swapaxes(codebook, 1, 2)
    out = pl.pallas_call(
        functools.partial(_rvq_voco_body, q_stages=Q),
        grid=(BN // TM,),
        in_specs=[
            pl.BlockSpec((TM, D), lambda i: (i, 0)),
            pl.BlockSpec((Q, K, D), lambda i: (0, 0, 0)),
            pl.BlockSpec((Q, D, K), lambda i: (0, 0, 0)),
            pl.BlockSpec((D, DF), lambda i: (0, 0)),
            pl.BlockSpec((1, DF), lambda i: (0, 0)),
            pl.BlockSpec((DF, HOP), lambda i: (0, 0)),
            pl.BlockSpec((1, HOP), lambda i: (0, 0)),
        ],
        out_specs=pl.BlockSpec((TM, HOP), lambda i: (i, 0)),
        out_shape=jax.ShapeDtypeStruct((BN, HOP), jnp.float32),
        scratch_shapes=[
            pltpu.VMEM((Q, K), jnp.float32),
            pltpu.VMEM((Q, K, 3 * D), jnp.bfloat16),
        ],
    )(x, codebook, cbt, W1, b1.reshape(1, DF), W2, b2.reshape(1, HOP))
    return out.reshape(B, N * HOP)


# speedup vs baseline: 3.2897x; 1.0054x over previous
import functools

import jax
import jax.numpy as jnp
from jax.experimental import pallas as pl
from jax.experimental.pallas import tpu as pltpu


def _rvq_stages(x, cbt_ref, csq_ref, hml_ref, q_stages, lane):
    d = x.shape[1]
    res = x
    qout = jnp.zeros_like(x)
    for q in range(q_stages):
        r_sq = jnp.sum(res * res, axis=1, keepdims=True)
        cross2 = jax.lax.dot_general(
            res.astype(jnp.bfloat16), cbt_ref[q], (((1,), (0,)), ((), ())),
            preferred_element_type=jnp.float32)
        dist = (r_sq - cross2) + csq_ref[q:q + 1, :]
        code = jnp.argmin(dist, axis=1)
        sel = (lane == code[:, None]).astype(jnp.bfloat16)
        q3 = jax.lax.dot_general(
            sel, hml_ref[q], (((1,), (0,)), ((), ())),
            preferred_element_type=jnp.float32)
        quant = (q3[:, :d] + q3[:, d:2 * d]) + q3[:, 2 * d:]
        qout = qout + quant
        res = res - quant
    return x + (qout - x)


def _rvq_voco_body(x_ref, cb_ref, cbt_ref, w1_ref, b1_ref, w2_ref, b2_ref,
                   out_ref, csq_ref, hml_ref, *, q_stages):
    @pl.when(pl.program_id(0) == 0)
    def _precompute():
        ones_d = jnp.ones((1, x_ref.shape[1]), dtype=jnp.float32)
        for q in range(q_stages):
            cb = cb_ref[q]
            csq_ref[q:q + 1, :] = jax.lax.dot_general(
                ones_d, cb * cb, (((1,), (1,)), ((), ())),
                preferred_element_type=jnp.float32,
                precision=jax.lax.Precision.HIGHEST)
            hi = cb.astype(jnp.bfloat16)
            r1 = cb - hi.astype(jnp.float32)
            mid = r1.astype(jnp.bfloat16)
            r2 = r1 - mid.astype(jnp.float32)
            hml_ref[q] = jnp.concatenate(
                [hi, mid, r2.astype(jnp.bfloat16)], axis=1)

    x = x_ref[...]
    tm = x.shape[0]
    lane = jax.lax.broadcasted_iota(jnp.int32, (tm, csq_ref.shape[1]), 1)
    qout = _rvq_stages(x, cbt_ref, csq_ref, hml_ref, q_stages, lane)
    feat = jnp.dot(qout, w1_ref[...], preferred_element_type=jnp.float32)
    feat = jax.nn.gelu(feat + b1_ref[...])
    frames = jnp.dot(feat, w2_ref[...], preferred_element_type=jnp.float32)
    out_ref[...] = frames + b2_ref[...]


def kernel(latents, codebook, W1, b1, W2, b2):
    B, N, D = latents.shape
    Q, K, _ = codebook.shape
    DF = W1.shape[1]
    HOP = W2.shape[1]
    BN = B * N
    TM = 1024
    x = latents.reshape(BN, D)
    cbt = (2.0 * jnp.swapaxes(codebook, 1, 2)).astype(jnp.bfloat16)
    out = pl.pallas_call(
        functools.partial(_rvq_voco_body, q_stages=Q),
        grid=(BN // TM,),
        in_specs=[
            pl.BlockSpec((TM, D), lambda i: (i, 0)),
            pl.BlockSpec((Q, K, D), lambda i: (0, 0, 0)),
            pl.BlockSpec((Q, D, K), lambda i: (0, 0, 0)),
            pl.BlockSpec((D, DF), lambda i: (0, 0)),
            pl.BlockSpec((1, DF), lambda i: (0, 0)),
            pl.BlockSpec((DF, HOP), lambda i: (0, 0)),
            pl.BlockSpec((1, HOP), lambda i: (0, 0)),
        ],
        out_specs=pl.BlockSpec((TM, HOP), lambda i: (i, 0)),
        out_shape=jax.ShapeDtypeStruct((BN, HOP), jnp.float32),
        scratch_shapes=[
            pltpu.VMEM((Q, K), jnp.float32),
            pltpu.VMEM((Q, K, 3 * D), jnp.bfloat16),
        ],
    )(x, codebook, cbt, W1, b1.reshape(1, DF), W2, b2.reshape(1, HOP))
    return out.reshape(B, N * HOP)
